# baseline (speedup 1.0000x reference)
import functools

import jax
import jax.numpy as jnp
from jax.experimental import pallas as pl
from jax.experimental.pallas import tpu as pltpu

_TBLK = 512
_K = 2048
_W = 64


def _vq_kernel(x_ref, cb_ref, cbt_ref, tok_ref, deq_ref, s1_ref, s2_ref,
               fit_ref, com_ref):
    x = x_ref[0]
    cb = cb_ref[...]
    cbt = cbt_ref[...]

    scores = jax.lax.dot_general(
        cb, x, (((1,), (0,)), ((), ())),
        preferred_element_type=jnp.float32,
        precision=jax.lax.Precision.DEFAULT)
    xn = jnp.sum(x * x, axis=0, keepdims=True)
    cn = jnp.sum(cb * cb, axis=1, keepdims=True)
    dist = xn - 2.0 * scores + cn

    mind = jnp.min(dist, axis=0, keepdims=True)
    kiota = jax.lax.broadcasted_iota(jnp.int32, dist.shape, 0).astype(jnp.float32)
    tokf = jnp.min(jnp.where(dist <= mind, kiota, float(_K)), axis=0,
                   keepdims=True)
    tok = tokf.astype(jnp.int32)

    onehot = (kiota == tokf).astype(jnp.float32)
    deq = jax.lax.dot_general(
        cbt, onehot, (((1,), (0,)), ((), ())),
        preferred_element_type=jnp.float32,
        precision=jax.lax.Precision.HIGHEST)

    tok_ref[0, 0] = tok
    deq_ref[0] = deq

    @pl.when((pl.program_id(0) == 0) & (pl.program_id(1) == 0))
    def _init():
        s1_ref[...] = jnp.zeros_like(s1_ref)
        s2_ref[...] = jnp.zeros_like(s2_ref)
        fit_ref[...] = jnp.zeros_like(fit_ref)
        com_ref[...] = jnp.zeros_like(com_ref)

    d = deq - x
    s1_ref[...] += jnp.sum(x).reshape(1, 1)
    s2_ref[...] += jnp.sum(x * x).reshape(1, 1)
    fit_ref[...] += jnp.sum(mind).reshape(1, 1)
    com_ref[...] += jnp.sum(d * d).reshape(1, 1)


@functools.partial(jax.jit, static_argnames=())
def kernel(hidden_states, codebook):
    B, W, S = hidden_states.shape
    K = codebook.shape[0]
    nt = S // _TBLK
    grid = (B, nt)

    scal = jax.ShapeDtypeStruct((1, 1), jnp.float32)
    scal_spec = pl.BlockSpec((1, 1), lambda b, t: (0, 0))

    tok4, deq, s1, s2, fit_s, com_s = pl.pallas_call(
        _vq_kernel,
        grid=grid,
        in_specs=[
            pl.BlockSpec((1, W, _TBLK), lambda b, t: (b, 0, t)),
            pl.BlockSpec((K, W), lambda b, t: (0, 0)),
            pl.BlockSpec((W, K), lambda b, t: (0, 0)),
        ],
        out_specs=[
            pl.BlockSpec((1, 1, 1, _TBLK), lambda b, t: (b, t, 0, 0)),
            pl.BlockSpec((1, W, _TBLK), lambda b, t: (b, 0, t)),
            scal_spec, scal_spec, scal_spec, scal_spec,
        ],
        out_shape=[
            jax.ShapeDtypeStruct((B, nt, 1, _TBLK), jnp.int32),
            jax.ShapeDtypeStruct((B, W, S), jnp.float32),
            scal, scal, scal, scal,
        ],
    )(hidden_states, codebook, codebook.T)

    n_total = float(B * W * S)
    n_rows = float(B * S)
    s1 = s1[0, 0]
    s2 = s2[0, 0]
    prenorm = jnp.sqrt(jnp.maximum(s2 - s1 * s1 / n_total, 0.0) / n_total)
    fit = fit_s[0, 0] / n_rows
    commit_loss = com_s[0, 0] / n_total
    music_tokens = tok4.reshape(B, S)
    return (music_tokens, deq, commit_loss, fit, prenorm)


# speedup vs baseline: 1.5873x; 1.5873x over previous
import functools

import jax
import jax.numpy as jnp
from jax import lax
from jax.experimental import pallas as pl
from jax.experimental.pallas import tpu as pltpu
from jax.experimental.pallas import tpu_sc as plsc

_TBLK = 512
_K = 2048
_W = 64

_NC = 2
_NS = 16
_NW = _NC * _NS
_CHUNK = 128


def _vq_kernel(x_ref, cb_ref, tok_ref, s1_ref, s2_ref, fit_ref):
    x = x_ref[0]
    cb = cb_ref[...]

    scores = jax.lax.dot_general(
        cb, x, (((1,), (0,)), ((), ())),
        preferred_element_type=jnp.float32,
        precision=jax.lax.Precision.DEFAULT)
    xn = jnp.sum(x * x, axis=0, keepdims=True)
    cn = jnp.sum(cb * cb, axis=1, keepdims=True)
    dist = xn - 2.0 * scores + cn

    mind = jnp.min(dist, axis=0, keepdims=True)
    mask = dist <= mind
    kiota = jax.lax.broadcasted_iota(jnp.int32, dist.shape, 0)
    tok = jnp.min(jnp.where(mask, kiota, _K), axis=0, keepdims=True)

    tok_ref[0, 0] = tok

    @pl.when((pl.program_id(0) == 0) & (pl.program_id(1) == 0))
    def _init():
        s1_ref[...] = jnp.zeros_like(s1_ref)
        s2_ref[...] = jnp.zeros_like(s2_ref)
        fit_ref[...] = jnp.zeros_like(fit_ref)

    s1_ref[...] += jnp.sum(x).reshape(1, 1)
    s2_ref[...] += jnp.sum(x * x).reshape(1, 1)
    fit_ref[...] += jnp.sum(mind).reshape(1, 1)


def _sc_gather(table_hbm, idx_hbm, out_hbm, idx_v, rows_v, sem):
    wid = lax.axis_index("s") * _NC + lax.axis_index("c")
    n = idx_hbm.shape[0]
    b_per_w = n // _NW
    base = wid * b_per_w
    for j in range(b_per_w // _CHUNK):
        off = base + j * _CHUNK
        pltpu.sync_copy(idx_hbm.at[pl.ds(off, _CHUNK)], idx_v)
        pltpu.async_copy(table_hbm.at[idx_v], rows_v, sem).wait()
        pltpu.sync_copy(rows_v, out_hbm.at[pl.ds(off, _CHUNK)])


def kernel(hidden_states, codebook):
    B, W, S = hidden_states.shape
    K = codebook.shape[0]
    nt = S // _TBLK
    grid = (B, nt)

    scal = jax.ShapeDtypeStruct((1, 1), jnp.float32)
    scal_spec = pl.BlockSpec((1, 1), lambda b, t: (0, 0))

    tok4, s1, s2, fit_s = pl.pallas_call(
        _vq_kernel,
        grid=grid,
        in_specs=[
            pl.BlockSpec((1, W, _TBLK), lambda b, t: (b, 0, t)),
            pl.BlockSpec((K, W), lambda b, t: (0, 0)),
        ],
        out_specs=[
            pl.BlockSpec((1, 1, 1, _TBLK), lambda b, t: (b, t, 0, 0)),
            scal_spec, scal_spec, scal_spec,
        ],
        out_shape=[
            jax.ShapeDtypeStruct((B, nt, 1, _TBLK), jnp.int32),
            scal, scal, scal,
        ],
    )(hidden_states, codebook)

    idx = tok4.reshape(B * S)
    table = jnp.pad(codebook, ((0, 0), (0, 128 - W)))
    mesh = plsc.VectorSubcoreMesh(core_axis_name="c", subcore_axis_name="s")
    rows = pl.kernel(
        _sc_gather,
        out_type=jax.ShapeDtypeStruct((B * S, 128), jnp.float32),
        mesh=mesh,
        scratch_types=[
            pltpu.VMEM((_CHUNK,), jnp.int32),
            pltpu.VMEM((_CHUNK, 128), jnp.float32),
            pltpu.SemaphoreType.DMA,
        ],
    )(table, idx)

    n_total = float(B * W * S)
    n_rows = float(B * S)
    s1 = s1[0, 0]
    s2 = s2[0, 0]
    prenorm = jnp.sqrt(jnp.maximum(s2 - s1 * s1 / n_total, 0.0) / n_total)
    fit = fit_s[0, 0] / n_rows
    commit_loss = fit_s[0, 0] / n_total
    music_tokens = tok4.reshape(B, S)
    deq = jnp.transpose(rows.reshape(B, S, 128)[:, :, :W], (0, 2, 1))
    return (music_tokens, deq, commit_loss, fit, prenorm)


# speedup vs baseline: 2.4713x; 1.5570x over previous
import functools

import jax
import jax.numpy as jnp
from jax.experimental import pallas as pl
from jax.experimental.pallas import tpu as pltpu

_TBLK = 1024
_K = 2048
_W = 64


def _vq_kernel(x_ref, cb_ref, cbt_ref, tok_ref, deq_ref, s1_ref, s2_ref,
               fit_ref, com_ref):
    x = x_ref[0]
    cb = cb_ref[...]
    cbt = cbt_ref[...]

    scores = jax.lax.dot_general(
        cb, x, (((1,), (0,)), ((), ())),
        preferred_element_type=jnp.float32,
        precision=jax.lax.Precision.DEFAULT)
    xn = jnp.sum(x * x, axis=0, keepdims=True)
    cn = jnp.sum(cb * cb, axis=1, keepdims=True)
    dist = xn - 2.0 * scores + cn

    mind = jnp.min(dist, axis=0, keepdims=True)
    mask = dist <= mind
    kiota = jax.lax.broadcasted_iota(jnp.int32, dist.shape, 0)
    tok = jnp.min(jnp.where(mask, kiota, _K), axis=0, keepdims=True)

    onehot = jnp.where(mask, 1.0, 0.0)
    deq = jax.lax.dot_general(
        cbt, onehot, (((1,), (0,)), ((), ())),
        preferred_element_type=jnp.float32,
        precision=jax.lax.Precision.DEFAULT)

    tok_ref[0, 0] = tok
    deq_ref[0] = deq

    @pl.when((pl.program_id(0) == 0) & (pl.program_id(1) == 0))
    def _init():
        s1_ref[...] = jnp.zeros_like(s1_ref)
        s2_ref[...] = jnp.zeros_like(s2_ref)
        fit_ref[...] = jnp.zeros_like(fit_ref)
        com_ref[...] = jnp.zeros_like(com_ref)

    d = deq - x
    s1_ref[...] += jnp.sum(x).reshape(1, 1)
    s2_ref[...] += jnp.sum(x * x).reshape(1, 1)
    fit_ref[...] += jnp.sum(mind).reshape(1, 1)
    com_ref[...] += jnp.sum(d * d).reshape(1, 1)


@functools.partial(jax.jit, static_argnames=())
def kernel(hidden_states, codebook):
    B, W, S = hidden_states.shape
    K = codebook.shape[0]
    nt = S // _TBLK
    grid = (B, nt)

    scal = jax.ShapeDtypeStruct((1, 1), jnp.float32)
    scal_spec = pl.BlockSpec((1, 1), lambda b, t: (0, 0))

    tok4, deq, s1, s2, fit_s, com_s = pl.pallas_call(
        _vq_kernel,
        grid=grid,
        in_specs=[
            pl.BlockSpec((1, W, _TBLK), lambda b, t: (b, 0, t)),
            pl.BlockSpec((K, W), lambda b, t: (0, 0)),
            pl.BlockSpec((W, K), lambda b, t: (0, 0)),
        ],
        out_specs=[
            pl.BlockSpec((1, 1, 1, _TBLK), lambda b, t: (b, t, 0, 0)),
            pl.BlockSpec((1, W, _TBLK), lambda b, t: (b, 0, t)),
            scal_spec, scal_spec, scal_spec, scal_spec,
        ],
        out_shape=[
            jax.ShapeDtypeStruct((B, nt, 1, _TBLK), jnp.int32),
            jax.ShapeDtypeStruct((B, W, S), jnp.float32),
            scal, scal, scal, scal,
        ],
    )(hidden_states, codebook, codebook.T)

    n_total = float(B * W * S)
    n_rows = float(B * S)
    s1 = s1[0, 0]
    s2 = s2[0, 0]
    prenorm = jnp.sqrt(jnp.maximum(s2 - s1 * s1 / n_total, 0.0) / n_total)
    fit = fit_s[0, 0] / n_rows
    commit_loss = com_s[0, 0] / n_total
    music_tokens = tok4.reshape(B, S)
    return (music_tokens, deq, commit_loss, fit, prenorm)
